# baseline (speedup 1.0000x reference)
import functools

import jax
import jax.numpy as jnp
from jax import lax
from jax.experimental import pallas as pl
from jax.experimental.pallas import tpu as pltpu
from jax.experimental.pallas import tpu_sc as plsc

_NC = 2
_NS = 16
_NW = _NC * _NS
_CNT_W = 16


def _proj_body(obj_ref, w1s_ref, w1o_ref, ps_ref, po_ref):
    x = obj_ref[...]
    ps_ref[...] = jnp.dot(x, w1s_ref[...], preferred_element_type=jnp.float32)
    po_ref[...] = jnp.dot(x, w1o_ref[...], preferred_element_type=jnp.float32)


def _tc_proj(obj, w1s_t, w1o_t):
    o, din = obj.shape
    h = w1s_t.shape[1]
    bo = 2000
    return pl.pallas_call(
        _proj_body,
        grid=(o // bo,),
        in_specs=[
            pl.BlockSpec((bo, din), lambda i: (i, 0)),
            pl.BlockSpec((din, h), lambda i: (0, 0)),
            pl.BlockSpec((din, h), lambda i: (0, 0)),
        ],
        out_specs=[pl.BlockSpec((bo, h), lambda i: (i, 0))] * 2,
        out_shape=[jax.ShapeDtypeStruct((o, h), jnp.float32)] * 2,
    )(obj, w1s_t, w1o_t)


def _edge_body(gs_ref, go_ref, pred_ref, w1p_ref, b1_ref, w2_ref, b2_ref,
               ns_ref, np_ref, no_ref):
    h = jnp.dot(pred_ref[...], w1p_ref[...], preferred_element_type=jnp.float32)
    h = jnp.maximum(h + gs_ref[...] + go_ref[...] + b1_ref[...], 0.0)
    t = jnp.dot(h, w2_ref[...], preferred_element_type=jnp.float32)
    t = jnp.maximum(t + b2_ref[...], 0.0)
    hh = ns_ref.shape[1]
    dout = np_ref.shape[1]
    ns_ref[...] = t[:, :hh]
    np_ref[...] = t[:, hh:hh + dout]
    no_ref[...] = t[:, hh + dout:]


def _tc_edge_mlp(g_s, g_o, pred, w1p_t, b1r, w2_t, b2r):
    t, din = pred.shape
    h = g_s.shape[1]
    dout2 = w2_t.shape[1]
    dout = dout2 - 2 * h
    be = 2000
    return pl.pallas_call(
        _edge_body,
        grid=(t // be,),
        in_specs=[
            pl.BlockSpec((be, h), lambda i: (i, 0)),
            pl.BlockSpec((be, h), lambda i: (i, 0)),
            pl.BlockSpec((be, din), lambda i: (i, 0)),
            pl.BlockSpec((din, h), lambda i: (0, 0)),
            pl.BlockSpec((1, h), lambda i: (0, 0)),
            pl.BlockSpec((h, dout2), lambda i: (0, 0)),
            pl.BlockSpec((1, dout2), lambda i: (0, 0)),
        ],
        out_specs=[
            pl.BlockSpec((be, h), lambda i: (i, 0)),
            pl.BlockSpec((be, dout), lambda i: (i, 0)),
            pl.BlockSpec((be, h), lambda i: (i, 0)),
        ],
        out_shape=[
            jax.ShapeDtypeStruct((t, h), jnp.float32),
            jax.ShapeDtypeStruct((t, dout), jnp.float32),
            jax.ShapeDtypeStruct((t, h), jnp.float32),
        ],
    )(g_s, g_o, pred, w1p_t, b1r, w2_t, b2r)


def _final_body(*refs):
    n_parts = (len(refs) - 5) // 2
    pp_refs = refs[:n_parts]
    cc_refs = refs[n_parts:2 * n_parts]
    w3_ref, b3_ref, w4_ref, b4_ref, out_ref = refs[2 * n_parts:]
    p = pp_refs[0][0] + pp_refs[0][1]
    c = cc_refs[0][0, :, 0:1] + cc_refs[0][1, :, 0:1]
    for k in range(1, n_parts):
        p = p + pp_refs[k][0] + pp_refs[k][1]
        c = c + cc_refs[k][0, :, 0:1] + cc_refs[k][1, :, 0:1]
    p = p / jnp.maximum(c, 1.0)
    h = jnp.dot(p, w3_ref[...], preferred_element_type=jnp.float32)
    h = jnp.maximum(h + b3_ref[...], 0.0)
    y = jnp.dot(h, w4_ref[...], preferred_element_type=jnp.float32)
    out_ref[...] = jnp.maximum(y + b4_ref[...], 0.0)


def _tc_final(pooled_list, cnt_list, w3_t, b3r, w4_t, b4r, o):
    h = pooled_list[0].shape[2]
    dout = w4_t.shape[1]
    bo = 2000
    n_parts = len(pooled_list)
    return pl.pallas_call(
        _final_body,
        grid=(o // bo,),
        in_specs=(
            [pl.BlockSpec((2, bo, h), lambda i: (0, i, 0))] * n_parts
            + [pl.BlockSpec((2, bo, _CNT_W), lambda i: (0, i, 0))] * n_parts
            + [
                pl.BlockSpec((h, h), lambda i: (0, 0)),
                pl.BlockSpec((1, h), lambda i: (0, 0)),
                pl.BlockSpec((h, dout), lambda i: (0, 0)),
                pl.BlockSpec((1, dout), lambda i: (0, 0)),
            ]
        ),
        out_specs=pl.BlockSpec((bo, dout), lambda i: (i, 0)),
        out_shape=jax.ShapeDtypeStruct((o, dout), jnp.float32),
    )(*pooled_list, *cnt_list, w3_t, b3r, w4_t, b4r)


def _pad_rows(o):
    return -(-o // (_NS * 128)) * (_NS * 128)


def _sc_gather(p_s, p_o, s_idx, o_idx, chunk):
    o, h = p_s.shape
    t = s_idx.shape[0]
    o_pad = _pad_rows(o)
    per_w = t // _NW
    n_chunks = per_w // chunk
    rows_per_tile = o_pad // _NS
    zrows = 128
    mesh = plsc.VectorSubcoreMesh(core_axis_name="c", subcore_axis_name="s")

    @functools.partial(
        pl.kernel,
        out_type=(
            jax.ShapeDtypeStruct((t, h), jnp.float32),
            jax.ShapeDtypeStruct((t, h), jnp.float32),
            jax.ShapeDtypeStruct((_NC, o_pad, _CNT_W), jnp.float32),
        ),
        mesh=mesh,
        compiler_params=pltpu.CompilerParams(use_tc_tiling_on_sc=False),
        scratch_types=(
            pltpu.VMEM((chunk,), jnp.int32),
            pltpu.VMEM((chunk,), jnp.int32),
            pltpu.VMEM((chunk, h), jnp.float32),
            pltpu.VMEM((chunk, h), jnp.float32),
            pltpu.VMEM((chunk, _CNT_W), jnp.float32),
            pltpu.VMEM((zrows, _CNT_W), jnp.float32),
            pltpu.VMEM_SHARED((o_pad, _CNT_W), jnp.float32),
            pltpu.SemaphoreType.DMA,
            pltpu.SemaphoreType.DMA,
        ),
    )
    def gather_k(ps_hbm, po_hbm, sidx_hbm, oidx_hbm,
                 gs_hbm, go_hbm, cnt_hbm,
                 sidx_v, oidx_v, rows_s, rows_o, ones_v, zeros_v, cnt_sh,
                 sem_s, sem_o):
        cid = lax.axis_index("c")
        sid = lax.axis_index("s")
        wid = sid * _NC + cid

        def fill_ones(i, carry):
            ones_v[i, :] = jnp.full((16,), 1.0, jnp.float32)
            return carry

        lax.fori_loop(0, chunk, fill_ones, 0)

        def fill_zeros(i, carry):
            zeros_v[i, :] = jnp.zeros((16,), jnp.float32)
            return carry

        lax.fori_loop(0, zrows, fill_zeros, 0)

        r0 = sid * rows_per_tile
        for z in range(rows_per_tile // zrows):
            pltpu.sync_copy(zeros_v, cnt_sh.at[pl.ds(r0 + z * zrows, zrows)])
        plsc.subcore_barrier()

        def chunk_body(ci, carry):
            base = wid * per_w + ci * chunk
            pltpu.sync_copy(sidx_hbm.at[pl.ds(base, chunk)], sidx_v)
            pltpu.sync_copy(oidx_hbm.at[pl.ds(base, chunk)], oidx_v)
            cp_s = pltpu.async_copy(ps_hbm.at[sidx_v], rows_s, sem_s)
            cp_o = pltpu.async_copy(po_hbm.at[oidx_v], rows_o, sem_o)
            cp_s.wait()
            cp_o.wait()
            pltpu.sync_copy(rows_s, gs_hbm.at[pl.ds(base, chunk)])
            pltpu.sync_copy(rows_o, go_hbm.at[pl.ds(base, chunk)])
            pltpu.sync_copy(ones_v, cnt_sh.at[sidx_v], add=True)
            pltpu.sync_copy(ones_v, cnt_sh.at[oidx_v], add=True)
            return carry

        lax.fori_loop(0, n_chunks, chunk_body, 0)

        plsc.subcore_barrier()
        pltpu.sync_copy(cnt_sh.at[pl.ds(r0, rows_per_tile)],
                        cnt_hbm.at[cid, pl.ds(r0, rows_per_tile)])

    return gather_k(p_s, p_o, s_idx, o_idx)


def _sc_scatter(new_s, new_o, s_idx, o_idx, o, chunk):
    t, h = new_s.shape
    o_pad = _pad_rows(o)
    per_w = t // _NW
    n_chunks = per_w // chunk
    rows_per_tile = o_pad // _NS
    zrows = 128
    mesh = plsc.VectorSubcoreMesh(core_axis_name="c", subcore_axis_name="s")

    @functools.partial(
        pl.kernel,
        out_type=jax.ShapeDtypeStruct((_NC, o_pad, h), jnp.float32),
        mesh=mesh,
        compiler_params=pltpu.CompilerParams(use_tc_tiling_on_sc=False),
        scratch_types=(
            pltpu.VMEM((chunk,), jnp.int32),
            pltpu.VMEM((chunk,), jnp.int32),
            pltpu.VMEM((chunk, h), jnp.float32),
            pltpu.VMEM((chunk, h), jnp.float32),
            pltpu.VMEM((zrows, h), jnp.float32),
            pltpu.VMEM_SHARED((o_pad, h), jnp.float32),
        ),
    )
    def scatter_k(ns_hbm, no_hbm, sidx_hbm, oidx_hbm, pooled_hbm,
                  sidx_v, oidx_v, rows_s, rows_o, zeros_v, pooled_sh):
        cid = lax.axis_index("c")
        sid = lax.axis_index("s")
        wid = sid * _NC + cid

        def fill_zeros(i, carry):
            for k in range(h // 16):
                zeros_v[i, pl.ds(k * 16, 16)] = jnp.zeros((16,), jnp.float32)
            return carry

        lax.fori_loop(0, zrows, fill_zeros, 0)

        r0 = sid * rows_per_tile
        for z in range(rows_per_tile // zrows):
            pltpu.sync_copy(zeros_v, pooled_sh.at[pl.ds(r0 + z * zrows, zrows)])
        plsc.subcore_barrier()

        def chunk_body(ci, carry):
            base = wid * per_w + ci * chunk
            pltpu.sync_copy(sidx_hbm.at[pl.ds(base, chunk)], sidx_v)
            pltpu.sync_copy(oidx_hbm.at[pl.ds(base, chunk)], oidx_v)
            pltpu.sync_copy(ns_hbm.at[pl.ds(base, chunk)], rows_s)
            pltpu.sync_copy(no_hbm.at[pl.ds(base, chunk)], rows_o)
            pltpu.sync_copy(rows_s, pooled_sh.at[sidx_v], add=True)
            pltpu.sync_copy(rows_o, pooled_sh.at[oidx_v], add=True)
            return carry

        lax.fori_loop(0, n_chunks, chunk_body, 0)

        plsc.subcore_barrier()
        pltpu.sync_copy(pooled_sh.at[pl.ds(r0, rows_per_tile)],
                        pooled_hbm.at[cid, pl.ds(r0, rows_per_tile)])

    return scatter_k(new_s, new_o, s_idx, o_idx)


def kernel(obj_vecs, pred_vecs, edges, W1, b1, W2, b2, W3, b3, W4, b4):
    o, din = obj_vecs.shape
    h = W1.shape[0]
    dout = W4.shape[0]

    s_idx = edges[:, 0]
    o_idx = edges[:, 1]
    w1s_t = W1[:, :din].T
    w1p_t = W1[:, din:2 * din].T
    w1o_t = W1[:, 2 * din:].T
    b1r = b1.reshape(1, h)
    w2_t = W2.T
    b2r = b2.reshape(1, -1)
    w3_t = W3.T
    b3r = b3.reshape(1, h)
    w4_t = W4.T
    b4r = b4.reshape(1, dout)

    p_s, p_o = _tc_proj(obj_vecs, w1s_t, w1o_t)

    t = s_idx.shape[0]
    t2 = t // 2
    chunk = 200
    parts = [(s_idx[:t2], o_idx[:t2], pred_vecs[:t2]),
             (s_idx[t2:], o_idx[t2:], pred_vecs[t2:])]

    gath = [_sc_gather(p_s, p_o, si, oi, chunk) for si, oi, _ in parts]
    mlps = [
        _tc_edge_mlp(g_s, g_o, pr, w1p_t, b1r, w2_t, b2r)
        for (g_s, g_o, _), (_, _, pr) in zip(gath, parts)
    ]
    pooled = [
        _sc_scatter(ns, no_, si, oi, o, chunk)
        for (ns, _, no_), (si, oi, _) in zip(mlps, parts)
    ]
    cnts = [g[2] for g in gath]
    new_obj = _tc_final(pooled, cnts, w3_t, b3r, w4_t, b4r, o)
    new_p = jnp.concatenate([m[1] for m in mlps], axis=0)
    return (new_obj, new_p)


# speedup vs baseline: 1.1573x; 1.1573x over previous
import functools

import jax
import jax.numpy as jnp
from jax import lax
from jax.experimental import pallas as pl
from jax.experimental.pallas import tpu as pltpu
from jax.experimental.pallas import tpu_sc as plsc

_NC = 2
_NS = 16
_NW = _NC * _NS
_CNT_W = 16


def _proj_body(obj_ref, w1s_ref, w1o_ref, ps_ref, po_ref):
    x = obj_ref[...]
    ps_ref[...] = jnp.dot(x, w1s_ref[...], preferred_element_type=jnp.float32)
    po_ref[...] = jnp.dot(x, w1o_ref[...], preferred_element_type=jnp.float32)


def _tc_proj(obj, w1s_t, w1o_t):
    o, din = obj.shape
    h = w1s_t.shape[1]
    bo = 2000
    return pl.pallas_call(
        _proj_body,
        grid=(o // bo,),
        in_specs=[
            pl.BlockSpec((bo, din), lambda i: (i, 0)),
            pl.BlockSpec((din, h), lambda i: (0, 0)),
            pl.BlockSpec((din, h), lambda i: (0, 0)),
        ],
        out_specs=[pl.BlockSpec((bo, h), lambda i: (i, 0))] * 2,
        out_shape=[jax.ShapeDtypeStruct((o, h), jnp.float32)] * 2,
    )(obj, w1s_t, w1o_t)


def _edge_body(gs_ref, go_ref, pred_ref, w1p_ref, b1_ref, w2_ref, b2_ref,
               ns_ref, np_ref, no_ref):
    h = jnp.dot(pred_ref[...], w1p_ref[...], preferred_element_type=jnp.float32)
    h = jnp.maximum(h + gs_ref[...] + go_ref[...] + b1_ref[...], 0.0)
    t = jnp.dot(h, w2_ref[...], preferred_element_type=jnp.float32)
    t = jnp.maximum(t + b2_ref[...], 0.0)
    hh = ns_ref.shape[1]
    dout = np_ref.shape[1]
    ns_ref[...] = t[:, :hh]
    np_ref[...] = t[:, hh:hh + dout]
    no_ref[...] = t[:, hh + dout:]


def _tc_edge_mlp(g_s, g_o, pred, w1p_t, b1r, w2_t, b2r):
    t, din = pred.shape
    h = g_s.shape[1]
    dout2 = w2_t.shape[1]
    dout = dout2 - 2 * h
    be = 2000
    return pl.pallas_call(
        _edge_body,
        grid=(t // be,),
        in_specs=[
            pl.BlockSpec((be, h), lambda i: (i, 0)),
            pl.BlockSpec((be, h), lambda i: (i, 0)),
            pl.BlockSpec((be, din), lambda i: (i, 0)),
            pl.BlockSpec((din, h), lambda i: (0, 0)),
            pl.BlockSpec((1, h), lambda i: (0, 0)),
            pl.BlockSpec((h, dout2), lambda i: (0, 0)),
            pl.BlockSpec((1, dout2), lambda i: (0, 0)),
        ],
        out_specs=[
            pl.BlockSpec((be, h), lambda i: (i, 0)),
            pl.BlockSpec((be, dout), lambda i: (i, 0)),
            pl.BlockSpec((be, h), lambda i: (i, 0)),
        ],
        out_shape=[
            jax.ShapeDtypeStruct((t, h), jnp.float32),
            jax.ShapeDtypeStruct((t, dout), jnp.float32),
            jax.ShapeDtypeStruct((t, h), jnp.float32),
        ],
    )(g_s, g_o, pred, w1p_t, b1r, w2_t, b2r)


def _final_body(*refs):
    n_parts = (len(refs) - 5) // 2
    pp_refs = refs[:n_parts]
    cc_refs = refs[n_parts:2 * n_parts]
    w3_ref, b3_ref, w4_ref, b4_ref, out_ref = refs[2 * n_parts:]
    p = pp_refs[0][0] + pp_refs[0][1]
    c = cc_refs[0][0, :, 0:1] + cc_refs[0][1, :, 0:1]
    for k in range(1, n_parts):
        p = p + pp_refs[k][0] + pp_refs[k][1]
        c = c + cc_refs[k][0, :, 0:1] + cc_refs[k][1, :, 0:1]
    p = p / jnp.maximum(c, 1.0)
    h = jnp.dot(p, w3_ref[...], preferred_element_type=jnp.float32)
    h = jnp.maximum(h + b3_ref[...], 0.0)
    y = jnp.dot(h, w4_ref[...], preferred_element_type=jnp.float32)
    out_ref[...] = jnp.maximum(y + b4_ref[...], 0.0)


def _tc_final(pooled_list, cnt_list, w3_t, b3r, w4_t, b4r, o):
    h = pooled_list[0].shape[2]
    dout = w4_t.shape[1]
    bo = 2000
    n_parts = len(pooled_list)
    return pl.pallas_call(
        _final_body,
        grid=(o // bo,),
        in_specs=(
            [pl.BlockSpec((2, bo, h), lambda i: (0, i, 0))] * n_parts
            + [pl.BlockSpec((2, bo, _CNT_W), lambda i: (0, i, 0))] * n_parts
            + [
                pl.BlockSpec((h, h), lambda i: (0, 0)),
                pl.BlockSpec((1, h), lambda i: (0, 0)),
                pl.BlockSpec((h, dout), lambda i: (0, 0)),
                pl.BlockSpec((1, dout), lambda i: (0, 0)),
            ]
        ),
        out_specs=pl.BlockSpec((bo, dout), lambda i: (i, 0)),
        out_shape=jax.ShapeDtypeStruct((o, dout), jnp.float32),
    )(*pooled_list, *cnt_list, w3_t, b3r, w4_t, b4r)


def _pad_rows(o):
    return -(-o // (_NS * 128)) * (_NS * 128)


def _sc_gather(p_s, p_o, s_idx, o_idx, chunk):
    o, h = p_s.shape
    t = s_idx.shape[0]
    o_pad = _pad_rows(o)
    per_w = t // _NW
    n_chunks = per_w // chunk
    rows_per_tile = o_pad // _NS
    zrows = 128
    mesh = plsc.VectorSubcoreMesh(core_axis_name="c", subcore_axis_name="s")

    nb = 2

    @functools.partial(
        pl.kernel,
        out_type=(
            jax.ShapeDtypeStruct((t, h), jnp.float32),
            jax.ShapeDtypeStruct((t, h), jnp.float32),
            jax.ShapeDtypeStruct((_NC, o_pad, _CNT_W), jnp.float32),
        ),
        mesh=mesh,
        compiler_params=pltpu.CompilerParams(use_tc_tiling_on_sc=False),
        scratch_types=(
            pltpu.VMEM((nb, chunk), jnp.int32),
            pltpu.VMEM((nb, chunk), jnp.int32),
            pltpu.VMEM((nb, chunk, h), jnp.float32),
            pltpu.VMEM((nb, chunk, h), jnp.float32),
            pltpu.VMEM((chunk, _CNT_W), jnp.float32),
            pltpu.VMEM((zrows, _CNT_W), jnp.float32),
            pltpu.VMEM_SHARED((o_pad, _CNT_W), jnp.float32),
            [pltpu.SemaphoreType.DMA] * nb,
            pltpu.SemaphoreType.DMA,
            [pltpu.SemaphoreType.DMA] * nb,
        ),
    )
    def gather_k(ps_hbm, po_hbm, sidx_hbm, oidx_hbm,
                 gs_hbm, go_hbm, cnt_hbm,
                 sidx_v, oidx_v, rows_s, rows_o,
                 ones_v, zeros_v, cnt_sh, sem_i, sem_g, sem_w):
        cid = lax.axis_index("c")
        sid = lax.axis_index("s")
        wid = sid * _NC + cid

        def fill_ones(i, carry):
            ones_v[i, :] = jnp.full((16,), 1.0, jnp.float32)
            return carry

        lax.fori_loop(0, chunk, fill_ones, 0)

        def fill_zeros(i, carry):
            zeros_v[i, :] = jnp.zeros((16,), jnp.float32)
            return carry

        lax.fori_loop(0, zrows, fill_zeros, 0)

        r0 = sid * rows_per_tile
        for z in range(rows_per_tile // zrows):
            pltpu.sync_copy(zeros_v, cnt_sh.at[pl.ds(r0 + z * zrows, zrows)])
        plsc.subcore_barrier()

        def idx_load(ci, b):
            base = wid * per_w + ci * chunk
            pltpu.async_copy(sidx_hbm.at[pl.ds(base, chunk)], sidx_v.at[b],
                             sem_i[b])
            pltpu.async_copy(oidx_hbm.at[pl.ds(base, chunk)], oidx_v.at[b],
                             sem_i[b])

        def wait_i(b):
            pltpu.make_async_copy(sidx_hbm.at[pl.ds(0, chunk)], sidx_v.at[b],
                                  sem_i[b]).wait()
            pltpu.make_async_copy(oidx_hbm.at[pl.ds(0, chunk)], oidx_v.at[b],
                                  sem_i[b]).wait()

        def wait_w(b):
            pltpu.make_async_copy(rows_s.at[b], gs_hbm.at[pl.ds(0, chunk)],
                                  sem_w[b]).wait()
            pltpu.make_async_copy(rows_o.at[b], go_hbm.at[pl.ds(0, chunk)],
                                  sem_w[b]).wait()

        for b in range(nb):
            idx_load(b, b)

        def chunk_step(ci, b):
            base = wid * per_w + ci * chunk
            wait_i(b)

            @pl.when(ci >= nb)
            def _():
                wait_w(b)

            d_s = pltpu.async_copy(ps_hbm.at[sidx_v.at[b]], rows_s.at[b],
                                   sem_g)
            d_o = pltpu.async_copy(po_hbm.at[oidx_v.at[b]], rows_o.at[b],
                                   sem_g)
            d_s.wait()
            d_o.wait()
            pltpu.sync_copy(ones_v, cnt_sh.at[sidx_v.at[b]], add=True)
            pltpu.sync_copy(ones_v, cnt_sh.at[oidx_v.at[b]], add=True)
            pltpu.async_copy(rows_s.at[b], gs_hbm.at[pl.ds(base, chunk)],
                             sem_w[b])
            pltpu.async_copy(rows_o.at[b], go_hbm.at[pl.ds(base, chunk)],
                             sem_w[b])
            ci2 = ci + nb
            ci2 = jnp.where(ci2 >= n_chunks, ci2 - n_chunks, ci2)
            idx_load(ci2, b)

        def chunk_body(ci, carry):
            for b in range(nb):
                @pl.when(lax.rem(ci, nb) == b)
                def _():
                    chunk_step(ci, b)
            return carry

        lax.fori_loop(0, n_chunks, chunk_body, 0)
        for b in range(nb):
            wait_i(b)
            wait_w(b)

        plsc.subcore_barrier()
        pltpu.sync_copy(cnt_sh.at[pl.ds(r0, rows_per_tile)],
                        cnt_hbm.at[cid, pl.ds(r0, rows_per_tile)])

    return gather_k(p_s, p_o, s_idx, o_idx)


def _sc_scatter(new_s, new_o, s_idx, o_idx, o, chunk):
    t, h = new_s.shape
    o_pad = _pad_rows(o)
    per_w = t // _NW
    n_chunks = per_w // chunk
    rows_per_tile = o_pad // _NS
    zrows = 128
    mesh = plsc.VectorSubcoreMesh(core_axis_name="c", subcore_axis_name="s")

    nb = 2

    @functools.partial(
        pl.kernel,
        out_type=jax.ShapeDtypeStruct((_NC, o_pad, h), jnp.float32),
        mesh=mesh,
        compiler_params=pltpu.CompilerParams(use_tc_tiling_on_sc=False),
        scratch_types=(
            pltpu.VMEM((nb, chunk), jnp.int32),
            pltpu.VMEM((nb, chunk), jnp.int32),
            pltpu.VMEM((nb, chunk, h), jnp.float32),
            pltpu.VMEM((nb, chunk, h), jnp.float32),
            pltpu.VMEM((zrows, h), jnp.float32),
            pltpu.VMEM_SHARED((o_pad, h), jnp.float32),
            [pltpu.SemaphoreType.DMA] * nb,
        ),
    )
    def scatter_k(ns_hbm, no_hbm, sidx_hbm, oidx_hbm, pooled_hbm,
                  sidx_v, oidx_v, rows_s, rows_o, zeros_v, pooled_sh,
                  sem_l):
        cid = lax.axis_index("c")
        sid = lax.axis_index("s")
        wid = sid * _NC + cid

        def fill_zeros(i, carry):
            for k in range(h // 16):
                zeros_v[i, pl.ds(k * 16, 16)] = jnp.zeros((16,), jnp.float32)
            return carry

        lax.fori_loop(0, zrows, fill_zeros, 0)

        r0 = sid * rows_per_tile
        for z in range(rows_per_tile // zrows):
            pltpu.sync_copy(zeros_v, pooled_sh.at[pl.ds(r0 + z * zrows, zrows)])
        plsc.subcore_barrier()

        def loads(ci, b):
            base = wid * per_w + ci * chunk
            pltpu.async_copy(sidx_hbm.at[pl.ds(base, chunk)], sidx_v.at[b],
                             sem_l[b])
            pltpu.async_copy(oidx_hbm.at[pl.ds(base, chunk)], oidx_v.at[b],
                             sem_l[b])
            pltpu.async_copy(ns_hbm.at[pl.ds(base, chunk)], rows_s.at[b],
                             sem_l[b])
            pltpu.async_copy(no_hbm.at[pl.ds(base, chunk)], rows_o.at[b],
                             sem_l[b])

        def wait_l(b):
            pltpu.make_async_copy(sidx_hbm.at[pl.ds(0, chunk)], sidx_v.at[b],
                                  sem_l[b]).wait()
            pltpu.make_async_copy(oidx_hbm.at[pl.ds(0, chunk)], oidx_v.at[b],
                                  sem_l[b]).wait()
            pltpu.make_async_copy(ns_hbm.at[pl.ds(0, chunk)], rows_s.at[b],
                                  sem_l[b]).wait()
            pltpu.make_async_copy(no_hbm.at[pl.ds(0, chunk)], rows_o.at[b],
                                  sem_l[b]).wait()

        for b in range(nb):
            loads(b, b)

        def chunk_step(ci, b):
            wait_l(b)
            pltpu.sync_copy(rows_s.at[b], pooled_sh.at[sidx_v.at[b]],
                            add=True)
            pltpu.sync_copy(rows_o.at[b], pooled_sh.at[oidx_v.at[b]],
                            add=True)
            ci2 = ci + nb
            ci2 = jnp.where(ci2 >= n_chunks, ci2 - n_chunks, ci2)
            loads(ci2, b)

        def chunk_body(ci, carry):
            for b in range(nb):
                @pl.when(lax.rem(ci, nb) == b)
                def _():
                    chunk_step(ci, b)
            return carry

        lax.fori_loop(0, n_chunks, chunk_body, 0)
        for b in range(nb):
            wait_l(b)

        plsc.subcore_barrier()
        pltpu.sync_copy(pooled_sh.at[pl.ds(r0, rows_per_tile)],
                        pooled_hbm.at[cid, pl.ds(r0, rows_per_tile)])

    return scatter_k(new_s, new_o, s_idx, o_idx)


def kernel(obj_vecs, pred_vecs, edges, W1, b1, W2, b2, W3, b3, W4, b4):
    o, din = obj_vecs.shape
    h = W1.shape[0]
    dout = W4.shape[0]

    s_idx = edges[:, 0]
    o_idx = edges[:, 1]
    w1s_t = W1[:, :din].T
    w1p_t = W1[:, din:2 * din].T
    w1o_t = W1[:, 2 * din:].T
    b1r = b1.reshape(1, h)
    w2_t = W2.T
    b2r = b2.reshape(1, -1)
    w3_t = W3.T
    b3r = b3.reshape(1, h)
    w4_t = W4.T
    b4r = b4.reshape(1, dout)

    p_s, p_o = _tc_proj(obj_vecs, w1s_t, w1o_t)
    g_s, g_o, cnt = _sc_gather(p_s, p_o, s_idx, o_idx, 400)
    new_s, new_p, new_o = _tc_edge_mlp(g_s, g_o, pred_vecs, w1p_t, b1r, w2_t, b2r)
    pooled = _sc_scatter(new_s, new_o, s_idx, o_idx, o, 200)
    new_obj = _tc_final([pooled], [cnt], w3_t, b3r, w4_t, b4r, o)
    return (new_obj, new_p)


# speedup vs baseline: 1.3540x; 1.1700x over previous
import functools

import jax
import jax.numpy as jnp
from jax import lax
from jax.experimental import pallas as pl
from jax.experimental.pallas import tpu as pltpu
from jax.experimental.pallas import tpu_sc as plsc

_NC = 2
_NS = 16
_NW = _NC * _NS
_CNT_W = 16


def _proj_body(obj_ref, w1s_ref, w1o_ref, ps_ref, po_ref):
    x = obj_ref[...]
    ps_ref[...] = jnp.dot(x, w1s_ref[...], preferred_element_type=jnp.float32)
    po_ref[...] = jnp.dot(x, w1o_ref[...], preferred_element_type=jnp.float32)


def _tc_proj(obj, w1s_t, w1o_t):
    o, din = obj.shape
    h = w1s_t.shape[1]
    bo = 2000
    return pl.pallas_call(
        _proj_body,
        grid=(o // bo,),
        in_specs=[
            pl.BlockSpec((bo, din), lambda i: (i, 0)),
            pl.BlockSpec((din, h), lambda i: (0, 0)),
            pl.BlockSpec((din, h), lambda i: (0, 0)),
        ],
        out_specs=[pl.BlockSpec((bo, h), lambda i: (i, 0))] * 2,
        out_shape=[jax.ShapeDtypeStruct((o, h), jnp.float32)] * 2,
    )(obj, w1s_t, w1o_t)


def _edge_body(g_ref, pred_ref, w1p_ref, b1_ref, w2_ref, b2_ref,
               ns_ref, np_ref, no_ref):
    h = jnp.dot(pred_ref[...], w1p_ref[...], preferred_element_type=jnp.float32)
    h = jnp.maximum(h + g_ref[...] + b1_ref[...], 0.0)
    t = jnp.dot(h, w2_ref[...], preferred_element_type=jnp.float32)
    t = jnp.maximum(t + b2_ref[...], 0.0)
    hh = ns_ref.shape[1]
    dout = np_ref.shape[1]
    ns_ref[...] = t[:, :hh]
    np_ref[...] = t[:, hh:hh + dout]
    no_ref[...] = t[:, hh + dout:]


def _tc_edge_mlp(g, pred, w1p_t, b1r, w2_t, b2r):
    t, din = pred.shape
    h = g.shape[1]
    dout2 = w2_t.shape[1]
    dout = dout2 - 2 * h
    be = 2000
    return pl.pallas_call(
        _edge_body,
        grid=(t // be,),
        in_specs=[
            pl.BlockSpec((be, h), lambda i: (i, 0)),
            pl.BlockSpec((be, din), lambda i: (i, 0)),
            pl.BlockSpec((din, h), lambda i: (0, 0)),
            pl.BlockSpec((1, h), lambda i: (0, 0)),
            pl.BlockSpec((h, dout2), lambda i: (0, 0)),
            pl.BlockSpec((1, dout2), lambda i: (0, 0)),
        ],
        out_specs=[
            pl.BlockSpec((be, h), lambda i: (i, 0)),
            pl.BlockSpec((be, dout), lambda i: (i, 0)),
            pl.BlockSpec((be, h), lambda i: (i, 0)),
        ],
        out_shape=[
            jax.ShapeDtypeStruct((t, h), jnp.float32),
            jax.ShapeDtypeStruct((t, dout), jnp.float32),
            jax.ShapeDtypeStruct((t, h), jnp.float32),
        ],
    )(g, pred, w1p_t, b1r, w2_t, b2r)


def _final_body(*refs):
    n_parts = (len(refs) - 5) // 2
    pp_refs = refs[:n_parts]
    cc_refs = refs[n_parts:2 * n_parts]
    w3_ref, b3_ref, w4_ref, b4_ref, out_ref = refs[2 * n_parts:]
    p = pp_refs[0][0] + pp_refs[0][1]
    c = cc_refs[0][0, :, 0:1] + cc_refs[0][1, :, 0:1]
    for k in range(1, n_parts):
        p = p + pp_refs[k][0] + pp_refs[k][1]
        c = c + cc_refs[k][0, :, 0:1] + cc_refs[k][1, :, 0:1]
    p = p / jnp.maximum(c, 1.0)
    h = jnp.dot(p, w3_ref[...], preferred_element_type=jnp.float32)
    h = jnp.maximum(h + b3_ref[...], 0.0)
    y = jnp.dot(h, w4_ref[...], preferred_element_type=jnp.float32)
    out_ref[...] = jnp.maximum(y + b4_ref[...], 0.0)


def _tc_final(pooled_list, cnt_list, w3_t, b3r, w4_t, b4r, o):
    h = pooled_list[0].shape[2]
    dout = w4_t.shape[1]
    bo = 2000
    n_parts = len(pooled_list)
    return pl.pallas_call(
        _final_body,
        grid=(o // bo,),
        in_specs=(
            [pl.BlockSpec((2, bo, h), lambda i: (0, i, 0))] * n_parts
            + [pl.BlockSpec((2, bo, _CNT_W), lambda i: (0, i, 0))] * n_parts
            + [
                pl.BlockSpec((h, h), lambda i: (0, 0)),
                pl.BlockSpec((1, h), lambda i: (0, 0)),
                pl.BlockSpec((h, dout), lambda i: (0, 0)),
                pl.BlockSpec((1, dout), lambda i: (0, 0)),
            ]
        ),
        out_specs=pl.BlockSpec((bo, dout), lambda i: (i, 0)),
        out_shape=jax.ShapeDtypeStruct((o, dout), jnp.float32),
    )(*pooled_list, *cnt_list, w3_t, b3r, w4_t, b4r)


def _pad_rows(o):
    return -(-o // (_NS * 128)) * (_NS * 128)


def _sc_gather(p_s, p_o, s_idx, o_idx, chunk):
    o, h = p_s.shape
    t = s_idx.shape[0]
    o_pad = _pad_rows(o)
    per_w = t // _NW
    n_chunks = per_w // chunk
    rows_per_tile = o_pad // _NS
    zrows = 128
    mesh = plsc.VectorSubcoreMesh(core_axis_name="c", subcore_axis_name="s")

    nb = 2

    @functools.partial(
        pl.kernel,
        out_type=(
            jax.ShapeDtypeStruct((t, h), jnp.float32),
            jax.ShapeDtypeStruct((_NC, o_pad, _CNT_W), jnp.float32),
        ),
        mesh=mesh,
        compiler_params=pltpu.CompilerParams(use_tc_tiling_on_sc=False),
        scratch_types=(
            pltpu.VMEM((nb, chunk), jnp.int32),
            pltpu.VMEM((nb, chunk), jnp.int32),
            pltpu.VMEM((nb, chunk, h), jnp.float32),
            pltpu.VMEM((chunk, _CNT_W), jnp.float32),
            pltpu.VMEM((zrows, _CNT_W), jnp.float32),
            pltpu.VMEM_SHARED((o_pad, _CNT_W), jnp.float32),
            [pltpu.SemaphoreType.DMA] * nb,
            pltpu.SemaphoreType.DMA,
            [pltpu.SemaphoreType.DMA] * nb,
        ),
    )
    def gather_k(ps_hbm, po_hbm, sidx_hbm, oidx_hbm,
                 g_hbm, cnt_hbm,
                 sidx_v, oidx_v, rows_g,
                 ones_v, zeros_v, cnt_sh, sem_i, sem_g, sem_w):
        cid = lax.axis_index("c")
        sid = lax.axis_index("s")
        wid = sid * _NC + cid

        def fill_ones(i, carry):
            ones_v[i, :] = jnp.full((16,), 1.0, jnp.float32)
            return carry

        lax.fori_loop(0, chunk, fill_ones, 0)

        def fill_zeros(i, carry):
            zeros_v[i, :] = jnp.zeros((16,), jnp.float32)
            return carry

        lax.fori_loop(0, zrows, fill_zeros, 0)

        r0 = sid * rows_per_tile
        for z in range(rows_per_tile // zrows):
            pltpu.sync_copy(zeros_v, cnt_sh.at[pl.ds(r0 + z * zrows, zrows)])
        plsc.subcore_barrier()

        def idx_load(ci, b):
            base = wid * per_w + ci * chunk
            pltpu.async_copy(sidx_hbm.at[pl.ds(base, chunk)], sidx_v.at[b],
                             sem_i[b])
            pltpu.async_copy(oidx_hbm.at[pl.ds(base, chunk)], oidx_v.at[b],
                             sem_i[b])

        def wait_i(b):
            pltpu.make_async_copy(sidx_hbm.at[pl.ds(0, chunk)], sidx_v.at[b],
                                  sem_i[b]).wait()
            pltpu.make_async_copy(oidx_hbm.at[pl.ds(0, chunk)], oidx_v.at[b],
                                  sem_i[b]).wait()

        def wait_w(b):
            pltpu.make_async_copy(rows_g.at[b], g_hbm.at[pl.ds(0, chunk)],
                                  sem_w[b]).wait()

        for b in range(nb):
            idx_load(b, b)

        def chunk_step(ci, b):
            base = wid * per_w + ci * chunk
            wait_i(b)

            @pl.when(ci >= nb)
            def _():
                wait_w(b)

            d_s = pltpu.async_copy(ps_hbm.at[sidx_v.at[b]], rows_g.at[b],
                                   sem_g)
            d_s.wait()
            d_o = pltpu.async_copy(po_hbm.at[oidx_v.at[b]], rows_g.at[b],
                                   sem_g, add=True)
            d_o.wait()
            pltpu.sync_copy(ones_v, cnt_sh.at[sidx_v.at[b]], add=True)
            pltpu.sync_copy(ones_v, cnt_sh.at[oidx_v.at[b]], add=True)
            pltpu.async_copy(rows_g.at[b], g_hbm.at[pl.ds(base, chunk)],
                             sem_w[b])
            ci2 = ci + nb
            ci2 = jnp.where(ci2 >= n_chunks, ci2 - n_chunks, ci2)
            idx_load(ci2, b)

        def chunk_body(ci, carry):
            for b in range(nb):
                @pl.when(lax.rem(ci, nb) == b)
                def _():
                    chunk_step(ci, b)
            return carry

        lax.fori_loop(0, n_chunks, chunk_body, 0)
        for b in range(nb):
            wait_i(b)
            wait_w(b)

        plsc.subcore_barrier()
        pltpu.sync_copy(cnt_sh.at[pl.ds(r0, rows_per_tile)],
                        cnt_hbm.at[cid, pl.ds(r0, rows_per_tile)])

    return gather_k(p_s, p_o, s_idx, o_idx)


def _sc_scatter(new_s, new_o, s_idx, o_idx, o, chunk):
    t, h = new_s.shape
    o_pad = _pad_rows(o)
    per_w = t // _NW
    n_chunks = per_w // chunk
    rows_per_tile = o_pad // _NS
    zrows = 128
    mesh = plsc.VectorSubcoreMesh(core_axis_name="c", subcore_axis_name="s")

    nb = 2

    @functools.partial(
        pl.kernel,
        out_type=jax.ShapeDtypeStruct((_NC, o_pad, h), jnp.float32),
        mesh=mesh,
        compiler_params=pltpu.CompilerParams(use_tc_tiling_on_sc=False),
        scratch_types=(
            pltpu.VMEM((nb, chunk), jnp.int32),
            pltpu.VMEM((nb, chunk), jnp.int32),
            pltpu.VMEM((nb, chunk, h), jnp.float32),
            pltpu.VMEM((nb, chunk, h), jnp.float32),
            pltpu.VMEM((zrows, h), jnp.float32),
            pltpu.VMEM_SHARED((o_pad, h), jnp.float32),
            [pltpu.SemaphoreType.DMA] * nb,
        ),
    )
    def scatter_k(ns_hbm, no_hbm, sidx_hbm, oidx_hbm, pooled_hbm,
                  sidx_v, oidx_v, rows_s, rows_o, zeros_v, pooled_sh,
                  sem_l):
        cid = lax.axis_index("c")
        sid = lax.axis_index("s")
        wid = sid * _NC + cid

        def fill_zeros(i, carry):
            for k in range(h // 16):
                zeros_v[i, pl.ds(k * 16, 16)] = jnp.zeros((16,), jnp.float32)
            return carry

        lax.fori_loop(0, zrows, fill_zeros, 0)

        r0 = sid * rows_per_tile
        for z in range(rows_per_tile // zrows):
            pltpu.sync_copy(zeros_v, pooled_sh.at[pl.ds(r0 + z * zrows, zrows)])
        plsc.subcore_barrier()

        def loads(ci, b):
            base = wid * per_w + ci * chunk
            pltpu.async_copy(sidx_hbm.at[pl.ds(base, chunk)], sidx_v.at[b],
                             sem_l[b])
            pltpu.async_copy(oidx_hbm.at[pl.ds(base, chunk)], oidx_v.at[b],
                             sem_l[b])
            pltpu.async_copy(ns_hbm.at[pl.ds(base, chunk)], rows_s.at[b],
                             sem_l[b])
            pltpu.async_copy(no_hbm.at[pl.ds(base, chunk)], rows_o.at[b],
                             sem_l[b])

        def wait_l(b):
            pltpu.make_async_copy(sidx_hbm.at[pl.ds(0, chunk)], sidx_v.at[b],
                                  sem_l[b]).wait()
            pltpu.make_async_copy(oidx_hbm.at[pl.ds(0, chunk)], oidx_v.at[b],
                                  sem_l[b]).wait()
            pltpu.make_async_copy(ns_hbm.at[pl.ds(0, chunk)], rows_s.at[b],
                                  sem_l[b]).wait()
            pltpu.make_async_copy(no_hbm.at[pl.ds(0, chunk)], rows_o.at[b],
                                  sem_l[b]).wait()

        for b in range(nb):
            loads(b, b)

        def chunk_step(ci, b):
            wait_l(b)
            pltpu.sync_copy(rows_s.at[b], pooled_sh.at[sidx_v.at[b]],
                            add=True)
            pltpu.sync_copy(rows_o.at[b], pooled_sh.at[oidx_v.at[b]],
                            add=True)
            ci2 = ci + nb
            ci2 = jnp.where(ci2 >= n_chunks, ci2 - n_chunks, ci2)
            loads(ci2, b)

        def chunk_body(ci, carry):
            for b in range(nb):
                @pl.when(lax.rem(ci, nb) == b)
                def _():
                    chunk_step(ci, b)
            return carry

        lax.fori_loop(0, n_chunks, chunk_body, 0)
        for b in range(nb):
            wait_l(b)

        plsc.subcore_barrier()
        pltpu.sync_copy(pooled_sh.at[pl.ds(r0, rows_per_tile)],
                        pooled_hbm.at[cid, pl.ds(r0, rows_per_tile)])

    return scatter_k(new_s, new_o, s_idx, o_idx)


def kernel(obj_vecs, pred_vecs, edges, W1, b1, W2, b2, W3, b3, W4, b4):
    o, din = obj_vecs.shape
    h = W1.shape[0]
    dout = W4.shape[0]

    s_idx = edges[:, 0]
    o_idx = edges[:, 1]
    w1s_t = W1[:, :din].T
    w1p_t = W1[:, din:2 * din].T
    w1o_t = W1[:, 2 * din:].T
    b1r = b1.reshape(1, h)
    w2_t = W2.T
    b2r = b2.reshape(1, -1)
    w3_t = W3.T
    b3r = b3.reshape(1, h)
    w4_t = W4.T
    b4r = b4.reshape(1, dout)

    p_s, p_o = _tc_proj(obj_vecs, w1s_t, w1o_t)
    g, cnt = _sc_gather(p_s, p_o, s_idx, o_idx, 400)
    new_s, new_p, new_o = _tc_edge_mlp(g, pred_vecs, w1p_t, b1r, w2_t, b2r)
    pooled = _sc_scatter(new_s, new_o, s_idx, o_idx, o, 200)
    new_obj = _tc_final([pooled], [cnt], w3_t, b3r, w4_t, b4r, o)
    return (new_obj, new_p)


# speedup vs baseline: 1.3665x; 1.0092x over previous
import functools

import jax
import jax.numpy as jnp
from jax import lax
from jax.experimental import pallas as pl
from jax.experimental.pallas import tpu as pltpu
from jax.experimental.pallas import tpu_sc as plsc

_NC = 2
_NS = 16
_NW = _NC * _NS
_CNT_W = 16


def _proj_body(obj_ref, w1s_ref, w1o_ref, ps_ref, po_ref):
    x = obj_ref[...]
    ps_ref[...] = jnp.dot(x, w1s_ref[...], preferred_element_type=jnp.float32)
    po_ref[...] = jnp.dot(x, w1o_ref[...], preferred_element_type=jnp.float32)


def _tc_proj(obj, w1s_t, w1o_t):
    o, din = obj.shape
    h = w1s_t.shape[1]
    bo = 2048
    return pl.pallas_call(
        _proj_body,
        grid=(o // bo,),
        in_specs=[
            pl.BlockSpec((bo, din), lambda i: (i, 0)),
            pl.BlockSpec((din, h), lambda i: (0, 0)),
            pl.BlockSpec((din, h), lambda i: (0, 0)),
        ],
        out_specs=[pl.BlockSpec((bo, h), lambda i: (i, 0))] * 2,
        out_shape=[jax.ShapeDtypeStruct((o, h), jnp.float32)] * 2,
    )(obj, w1s_t, w1o_t)


def _edge_body(g_ref, pred_ref, w1p_ref, b1_ref, w2_ref, b2_ref,
               ns_ref, np_ref, no_ref):
    h = jnp.dot(pred_ref[...], w1p_ref[...], preferred_element_type=jnp.float32)
    h = jnp.maximum(h + g_ref[...] + b1_ref[...], 0.0)
    t = jnp.dot(h, w2_ref[...], preferred_element_type=jnp.float32)
    t = jnp.maximum(t + b2_ref[...], 0.0)
    hh = ns_ref.shape[1]
    dout = np_ref.shape[1]
    ns_ref[...] = t[:, :hh]
    np_ref[...] = t[:, hh:hh + dout]
    no_ref[...] = t[:, hh + dout:]


def _tc_edge_mlp(g, pred, w1p_t, b1r, w2_t, b2r):
    t, din = pred.shape
    h = g.shape[1]
    dout2 = w2_t.shape[1]
    dout = dout2 - 2 * h
    be = 2000
    return pl.pallas_call(
        _edge_body,
        grid=(t // be,),
        in_specs=[
            pl.BlockSpec((be, h), lambda i: (i, 0)),
            pl.BlockSpec((be, din), lambda i: (i, 0)),
            pl.BlockSpec((din, h), lambda i: (0, 0)),
            pl.BlockSpec((1, h), lambda i: (0, 0)),
            pl.BlockSpec((h, dout2), lambda i: (0, 0)),
            pl.BlockSpec((1, dout2), lambda i: (0, 0)),
        ],
        out_specs=[
            pl.BlockSpec((be, h), lambda i: (i, 0)),
            pl.BlockSpec((be, dout), lambda i: (i, 0)),
            pl.BlockSpec((be, h), lambda i: (i, 0)),
        ],
        out_shape=[
            jax.ShapeDtypeStruct((t, h), jnp.float32),
            jax.ShapeDtypeStruct((t, dout), jnp.float32),
            jax.ShapeDtypeStruct((t, h), jnp.float32),
        ],
    )(g, pred, w1p_t, b1r, w2_t, b2r)


def _final_body(*refs):
    n_parts = (len(refs) - 5) // 2
    pp_refs = refs[:n_parts]
    cc_refs = refs[n_parts:2 * n_parts]
    w3_ref, b3_ref, w4_ref, b4_ref, out_ref = refs[2 * n_parts:]
    p = pp_refs[0][0] + pp_refs[0][1]
    c = cc_refs[0][0, :, 0:1] + cc_refs[0][1, :, 0:1]
    for k in range(1, n_parts):
        p = p + pp_refs[k][0] + pp_refs[k][1]
        c = c + cc_refs[k][0, :, 0:1] + cc_refs[k][1, :, 0:1]
    p = p / jnp.maximum(c, 1.0)
    h = jnp.dot(p, w3_ref[...], preferred_element_type=jnp.float32)
    h = jnp.maximum(h + b3_ref[...], 0.0)
    y = jnp.dot(h, w4_ref[...], preferred_element_type=jnp.float32)
    out_ref[...] = jnp.maximum(y + b4_ref[...], 0.0)


def _tc_final(pooled_list, cnt_list, w3_t, b3r, w4_t, b4r, o):
    h = pooled_list[0].shape[2]
    dout = w4_t.shape[1]
    bo = 2000
    n_parts = len(pooled_list)
    return pl.pallas_call(
        _final_body,
        grid=(o // bo,),
        in_specs=(
            [pl.BlockSpec((2, bo, h), lambda i: (0, i, 0))] * n_parts
            + [pl.BlockSpec((2, bo, _CNT_W), lambda i: (0, i, 0))] * n_parts
            + [
                pl.BlockSpec((h, h), lambda i: (0, 0)),
                pl.BlockSpec((1, h), lambda i: (0, 0)),
                pl.BlockSpec((h, dout), lambda i: (0, 0)),
                pl.BlockSpec((1, dout), lambda i: (0, 0)),
            ]
        ),
        out_specs=pl.BlockSpec((bo, dout), lambda i: (i, 0)),
        out_shape=jax.ShapeDtypeStruct((o, dout), jnp.float32),
    )(*pooled_list, *cnt_list, w3_t, b3r, w4_t, b4r)


def _pad_rows(o):
    return -(-o // (_NS * 128)) * (_NS * 128)


def _sc_gather(p_s, p_o, s_idx, o_idx, chunk):
    o, h = p_s.shape
    t = s_idx.shape[0]
    o_pad = _pad_rows(o)
    per_w = t // _NW
    n_chunks = per_w // chunk
    rows_per_tile = o_pad // _NS
    zrows = 128
    mesh = plsc.VectorSubcoreMesh(core_axis_name="c", subcore_axis_name="s")

    nb = 2

    @functools.partial(
        pl.kernel,
        out_type=(
            jax.ShapeDtypeStruct((t, h), jnp.float32),
            jax.ShapeDtypeStruct((_NC, o_pad, _CNT_W), jnp.float32),
        ),
        mesh=mesh,
        compiler_params=pltpu.CompilerParams(use_tc_tiling_on_sc=False),
        scratch_types=(
            pltpu.VMEM((nb, chunk), jnp.int32),
            pltpu.VMEM((nb, chunk), jnp.int32),
            pltpu.VMEM((nb, chunk, h), jnp.float32),
            pltpu.VMEM((chunk, _CNT_W), jnp.float32),
            pltpu.VMEM((zrows, _CNT_W), jnp.float32),
            pltpu.VMEM_SHARED((o_pad, _CNT_W), jnp.float32),
            pltpu.VMEM_SHARED((o_pad, 64), jnp.float32),
            pltpu.VMEM_SHARED((o_pad, 64), jnp.float32),
            [pltpu.SemaphoreType.DMA] * nb,
            pltpu.SemaphoreType.DMA,
            [pltpu.SemaphoreType.DMA] * nb,
        ),
    )
    def gather_k(ps_hbm, po_hbm, sidx_hbm, oidx_hbm,
                 g_hbm, cnt_hbm,
                 sidx_v, oidx_v, rows_g,
                 ones_v, zeros_v, cnt_sh, ps_sh, po_sh, sem_i, sem_g, sem_w):
        cid = lax.axis_index("c")
        sid = lax.axis_index("s")
        wid = sid * _NC + cid

        def fill_ones(i, carry):
            ones_v[i, :] = jnp.full((16,), 1.0, jnp.float32)
            return carry

        lax.fori_loop(0, chunk, fill_ones, 0)

        def fill_zeros(i, carry):
            zeros_v[i, :] = jnp.zeros((16,), jnp.float32)
            return carry

        lax.fori_loop(0, zrows, fill_zeros, 0)

        r0 = sid * rows_per_tile
        tbl_s = pltpu.async_copy(ps_hbm.at[pl.ds(r0, rows_per_tile)],
                                 ps_sh.at[pl.ds(r0, rows_per_tile)], sem_g)
        tbl_o = pltpu.async_copy(po_hbm.at[pl.ds(r0, rows_per_tile)],
                                 po_sh.at[pl.ds(r0, rows_per_tile)], sem_g)
        for z in range(rows_per_tile // zrows):
            pltpu.sync_copy(zeros_v, cnt_sh.at[pl.ds(r0 + z * zrows, zrows)])
        tbl_s.wait()
        tbl_o.wait()
        plsc.subcore_barrier()

        def idx_load(ci, b):
            base = wid * per_w + ci * chunk
            pltpu.async_copy(sidx_hbm.at[pl.ds(base, chunk)], sidx_v.at[b],
                             sem_i[b])
            pltpu.async_copy(oidx_hbm.at[pl.ds(base, chunk)], oidx_v.at[b],
                             sem_i[b])

        def wait_i(b):
            pltpu.make_async_copy(sidx_hbm.at[pl.ds(0, chunk)], sidx_v.at[b],
                                  sem_i[b]).wait()
            pltpu.make_async_copy(oidx_hbm.at[pl.ds(0, chunk)], oidx_v.at[b],
                                  sem_i[b]).wait()

        def wait_w(b):
            pltpu.make_async_copy(rows_g.at[b], g_hbm.at[pl.ds(0, chunk)],
                                  sem_w[b]).wait()

        for b in range(nb):
            idx_load(b, b)

        def chunk_step(ci, b):
            base = wid * per_w + ci * chunk
            wait_i(b)

            @pl.when(ci >= nb)
            def _():
                wait_w(b)

            pltpu.sync_copy(ps_sh.at[sidx_v.at[b]], rows_g.at[b])
            pltpu.sync_copy(po_sh.at[oidx_v.at[b]], rows_g.at[b], add=True)
            pltpu.sync_copy(ones_v, cnt_sh.at[sidx_v.at[b]], add=True)
            pltpu.sync_copy(ones_v, cnt_sh.at[oidx_v.at[b]], add=True)
            pltpu.async_copy(rows_g.at[b], g_hbm.at[pl.ds(base, chunk)],
                             sem_w[b])
            ci2 = ci + nb
            ci2 = jnp.where(ci2 >= n_chunks, ci2 - n_chunks, ci2)
            idx_load(ci2, b)

        def chunk_body(ci, carry):
            for b in range(nb):
                @pl.when(lax.rem(ci, nb) == b)
                def _():
                    chunk_step(ci, b)
            return carry

        lax.fori_loop(0, n_chunks, chunk_body, 0)
        for b in range(nb):
            wait_i(b)
            wait_w(b)

        plsc.subcore_barrier()
        pltpu.sync_copy(cnt_sh.at[pl.ds(r0, rows_per_tile)],
                        cnt_hbm.at[cid, pl.ds(r0, rows_per_tile)])

    return gather_k(p_s, p_o, s_idx, o_idx)


def _sc_scatter(new_s, new_o, s_idx, o_idx, o, chunk):
    t, h = new_s.shape
    o_pad = _pad_rows(o)
    per_w = t // _NW
    n_chunks = per_w // chunk
    rows_per_tile = o_pad // _NS
    zrows = 128
    mesh = plsc.VectorSubcoreMesh(core_axis_name="c", subcore_axis_name="s")

    nb = 2

    @functools.partial(
        pl.kernel,
        out_type=jax.ShapeDtypeStruct((_NC, o_pad, h), jnp.float32),
        mesh=mesh,
        compiler_params=pltpu.CompilerParams(use_tc_tiling_on_sc=False),
        scratch_types=(
            pltpu.VMEM((nb, chunk), jnp.int32),
            pltpu.VMEM((nb, chunk), jnp.int32),
            pltpu.VMEM((nb, chunk, h), jnp.float32),
            pltpu.VMEM((nb, chunk, h), jnp.float32),
            pltpu.VMEM((zrows, h), jnp.float32),
            pltpu.VMEM_SHARED((o_pad, h), jnp.float32),
            [pltpu.SemaphoreType.DMA] * nb,
        ),
    )
    def scatter_k(ns_hbm, no_hbm, sidx_hbm, oidx_hbm, pooled_hbm,
                  sidx_v, oidx_v, rows_s, rows_o, zeros_v, pooled_sh,
                  sem_l):
        cid = lax.axis_index("c")
        sid = lax.axis_index("s")
        wid = sid * _NC + cid

        def fill_zeros(i, carry):
            for k in range(h // 16):
                zeros_v[i, pl.ds(k * 16, 16)] = jnp.zeros((16,), jnp.float32)
            return carry

        lax.fori_loop(0, zrows, fill_zeros, 0)

        r0 = sid * rows_per_tile
        for z in range(rows_per_tile // zrows):
            pltpu.sync_copy(zeros_v, pooled_sh.at[pl.ds(r0 + z * zrows, zrows)])
        plsc.subcore_barrier()

        def loads(ci, b):
            base = wid * per_w + ci * chunk
            pltpu.async_copy(sidx_hbm.at[pl.ds(base, chunk)], sidx_v.at[b],
                             sem_l[b])
            pltpu.async_copy(oidx_hbm.at[pl.ds(base, chunk)], oidx_v.at[b],
                             sem_l[b])
            pltpu.async_copy(ns_hbm.at[pl.ds(base, chunk)], rows_s.at[b],
                             sem_l[b])
            pltpu.async_copy(no_hbm.at[pl.ds(base, chunk)], rows_o.at[b],
                             sem_l[b])

        def wait_l(b):
            pltpu.make_async_copy(sidx_hbm.at[pl.ds(0, chunk)], sidx_v.at[b],
                                  sem_l[b]).wait()
            pltpu.make_async_copy(oidx_hbm.at[pl.ds(0, chunk)], oidx_v.at[b],
                                  sem_l[b]).wait()
            pltpu.make_async_copy(ns_hbm.at[pl.ds(0, chunk)], rows_s.at[b],
                                  sem_l[b]).wait()
            pltpu.make_async_copy(no_hbm.at[pl.ds(0, chunk)], rows_o.at[b],
                                  sem_l[b]).wait()

        for b in range(nb):
            loads(b, b)

        def chunk_step(ci, b):
            wait_l(b)
            pltpu.sync_copy(rows_s.at[b], pooled_sh.at[sidx_v.at[b]],
                            add=True)
            pltpu.sync_copy(rows_o.at[b], pooled_sh.at[oidx_v.at[b]],
                            add=True)
            ci2 = ci + nb
            ci2 = jnp.where(ci2 >= n_chunks, ci2 - n_chunks, ci2)
            loads(ci2, b)

        def chunk_body(ci, carry):
            for b in range(nb):
                @pl.when(lax.rem(ci, nb) == b)
                def _():
                    chunk_step(ci, b)
            return carry

        lax.fori_loop(0, n_chunks, chunk_body, 0)
        for b in range(nb):
            wait_l(b)

        plsc.subcore_barrier()
        pltpu.sync_copy(pooled_sh.at[pl.ds(r0, rows_per_tile)],
                        pooled_hbm.at[cid, pl.ds(r0, rows_per_tile)])

    return scatter_k(new_s, new_o, s_idx, o_idx)


def kernel(obj_vecs, pred_vecs, edges, W1, b1, W2, b2, W3, b3, W4, b4):
    o, din = obj_vecs.shape
    h = W1.shape[0]
    dout = W4.shape[0]

    s_idx = edges[:, 0]
    o_idx = edges[:, 1]
    w1s_t = W1[:, :din].T
    w1p_t = W1[:, din:2 * din].T
    w1o_t = W1[:, 2 * din:].T
    b1r = b1.reshape(1, h)
    w2_t = W2.T
    b2r = b2.reshape(1, -1)
    w3_t = W3.T
    b3r = b3.reshape(1, h)
    w4_t = W4.T
    b4r = b4.reshape(1, dout)

    o_pad = _pad_rows(o)
    obj_pad = jnp.pad(obj_vecs, ((0, o_pad - o), (0, 0)))
    p_s, p_o = _tc_proj(obj_pad, w1s_t, w1o_t)
    g, cnt = _sc_gather(p_s, p_o, s_idx, o_idx, 200)
    new_s, new_p, new_o = _tc_edge_mlp(g, pred_vecs, w1p_t, b1r, w2_t, b2r)
    pooled = _sc_scatter(new_s, new_o, s_idx, o_idx, o, 200)
    new_obj = _tc_final([pooled], [cnt], w3_t, b3r, w4_t, b4r, o)
    return (new_obj, new_p)


# speedup vs baseline: 1.4332x; 1.0489x over previous
import functools

import jax
import jax.numpy as jnp
from jax import lax
from jax.experimental import pallas as pl
from jax.experimental.pallas import tpu as pltpu
from jax.experimental.pallas import tpu_sc as plsc

_NC = 2
_NS = 16
_NW = _NC * _NS
_CNT_W = 16


def _proj_body(obj_ref, w1s_ref, w1o_ref, ps_ref, po_ref):
    x = obj_ref[...]
    ps_ref[...] = jnp.dot(x, w1s_ref[...], preferred_element_type=jnp.float32)
    po_ref[...] = jnp.dot(x, w1o_ref[...], preferred_element_type=jnp.float32)


def _tc_proj(obj, w1s_t, w1o_t):
    o, din = obj.shape
    h = w1s_t.shape[1]
    bo = 2048
    return pl.pallas_call(
        _proj_body,
        grid=(o // bo,),
        in_specs=[
            pl.BlockSpec((bo, din), lambda i: (i, 0)),
            pl.BlockSpec((din, h), lambda i: (0, 0)),
            pl.BlockSpec((din, h), lambda i: (0, 0)),
        ],
        out_specs=[pl.BlockSpec((bo, h), lambda i: (i, 0))] * 2,
        out_shape=[jax.ShapeDtypeStruct((o, h), jnp.float32)] * 2,
    )(obj, w1s_t, w1o_t)


def _edge_body(g_ref, pred_ref, w1p_ref, b1_ref, w2_ref, b2_ref,
               ns_ref, np_ref, no_ref):
    h = jnp.dot(pred_ref[...], w1p_ref[...], preferred_element_type=jnp.float32)
    h = jnp.maximum(h + g_ref[...] + b1_ref[...], 0.0)
    t = jnp.dot(h, w2_ref[...], preferred_element_type=jnp.float32)
    t = jnp.maximum(t + b2_ref[...], 0.0)
    hh = ns_ref.shape[1]
    dout = np_ref.shape[1]
    ns_ref[...] = t[:, :hh]
    np_ref[...] = t[:, hh:hh + dout]
    no_ref[...] = t[:, hh + dout:]


def _tc_edge_mlp(g, pred, w1p_t, b1r, w2_t, b2r):
    t, din = pred.shape
    h = g.shape[1]
    dout2 = w2_t.shape[1]
    dout = dout2 - 2 * h
    be = 5000
    return pl.pallas_call(
        _edge_body,
        grid=(t // be,),
        in_specs=[
            pl.BlockSpec((be, h), lambda i: (i, 0)),
            pl.BlockSpec((be, din), lambda i: (i, 0)),
            pl.BlockSpec((din, h), lambda i: (0, 0)),
            pl.BlockSpec((1, h), lambda i: (0, 0)),
            pl.BlockSpec((h, dout2), lambda i: (0, 0)),
            pl.BlockSpec((1, dout2), lambda i: (0, 0)),
        ],
        out_specs=[
            pl.BlockSpec((be, h), lambda i: (i, 0)),
            pl.BlockSpec((be, dout), lambda i: (i, 0)),
            pl.BlockSpec((be, h), lambda i: (i, 0)),
        ],
        out_shape=[
            jax.ShapeDtypeStruct((t, h), jnp.float32),
            jax.ShapeDtypeStruct((t, dout), jnp.float32),
            jax.ShapeDtypeStruct((t, h), jnp.float32),
        ],
    )(g, pred, w1p_t, b1r, w2_t, b2r)


def _final_body(*refs):
    n_parts = (len(refs) - 5) // 2
    pp_refs = refs[:n_parts]
    cc_refs = refs[n_parts:2 * n_parts]
    w3_ref, b3_ref, w4_ref, b4_ref, out_ref = refs[2 * n_parts:]
    p = pp_refs[0][0] + pp_refs[0][1]
    c = cc_refs[0][0, :, 0:1] + cc_refs[0][1, :, 0:1]
    for k in range(1, n_parts):
        p = p + pp_refs[k][0] + pp_refs[k][1]
        c = c + cc_refs[k][0, :, 0:1] + cc_refs[k][1, :, 0:1]
    p = p / jnp.maximum(c, 1.0)
    h = jnp.dot(p, w3_ref[...], preferred_element_type=jnp.float32)
    h = jnp.maximum(h + b3_ref[...], 0.0)
    y = jnp.dot(h, w4_ref[...], preferred_element_type=jnp.float32)
    out_ref[...] = jnp.maximum(y + b4_ref[...], 0.0)


def _tc_final(pooled_list, cnt_list, w3_t, b3r, w4_t, b4r, o):
    h = pooled_list[0].shape[2]
    dout = w4_t.shape[1]
    bo = 2000
    n_parts = len(pooled_list)
    return pl.pallas_call(
        _final_body,
        grid=(o // bo,),
        in_specs=(
            [pl.BlockSpec((2, bo, h), lambda i: (0, i, 0))] * n_parts
            + [pl.BlockSpec((2, bo, _CNT_W), lambda i: (0, i, 0))] * n_parts
            + [
                pl.BlockSpec((h, h), lambda i: (0, 0)),
                pl.BlockSpec((1, h), lambda i: (0, 0)),
                pl.BlockSpec((h, dout), lambda i: (0, 0)),
                pl.BlockSpec((1, dout), lambda i: (0, 0)),
            ]
        ),
        out_specs=pl.BlockSpec((bo, dout), lambda i: (i, 0)),
        out_shape=jax.ShapeDtypeStruct((o, dout), jnp.float32),
    )(*pooled_list, *cnt_list, w3_t, b3r, w4_t, b4r)


def _pad_rows(o):
    return -(-o // (_NS * 128)) * (_NS * 128)


def _sc_gather(p_s, p_o, s_idx, o_idx, chunk):
    o, h = p_s.shape
    t = s_idx.shape[0]
    o_pad = _pad_rows(o)
    per_w = t // _NW
    n_chunks = per_w // chunk
    rows_per_tile = o_pad // _NS
    zrows = 128
    mesh = plsc.VectorSubcoreMesh(core_axis_name="c", subcore_axis_name="s")

    nb = 2

    @functools.partial(
        pl.kernel,
        out_type=(
            jax.ShapeDtypeStruct((t, h), jnp.float32),
            jax.ShapeDtypeStruct((_NC, o_pad, _CNT_W), jnp.float32),
        ),
        mesh=mesh,
        compiler_params=pltpu.CompilerParams(use_tc_tiling_on_sc=False),
        scratch_types=(
            pltpu.VMEM((nb, chunk), jnp.int32),
            pltpu.VMEM((nb, chunk), jnp.int32),
            pltpu.VMEM((nb, chunk, h), jnp.float32),
            pltpu.VMEM((chunk, _CNT_W), jnp.float32),
            pltpu.VMEM((zrows, _CNT_W), jnp.float32),
            pltpu.VMEM_SHARED((o_pad, _CNT_W), jnp.float32),
            pltpu.VMEM_SHARED((o_pad, 64), jnp.float32),
            pltpu.VMEM_SHARED((o_pad, 64), jnp.float32),
            [pltpu.SemaphoreType.DMA] * nb,
            pltpu.SemaphoreType.DMA,
            [pltpu.SemaphoreType.DMA] * nb,
        ),
    )
    def gather_k(ps_hbm, po_hbm, sidx_hbm, oidx_hbm,
                 g_hbm, cnt_hbm,
                 sidx_v, oidx_v, rows_g,
                 ones_v, zeros_v, cnt_sh, ps_sh, po_sh, sem_i, sem_g, sem_w):
        cid = lax.axis_index("c")
        sid = lax.axis_index("s")
        wid = sid * _NC + cid

        def fill_ones(i, carry):
            ones_v[i, :] = jnp.full((16,), 1.0, jnp.float32)
            return carry

        lax.fori_loop(0, chunk, fill_ones, 0)

        def fill_zeros(i, carry):
            zeros_v[i, :] = jnp.zeros((16,), jnp.float32)
            return carry

        lax.fori_loop(0, zrows, fill_zeros, 0)

        r0 = sid * rows_per_tile
        tbl_s = pltpu.async_copy(ps_hbm.at[pl.ds(r0, rows_per_tile)],
                                 ps_sh.at[pl.ds(r0, rows_per_tile)], sem_g)
        tbl_o = pltpu.async_copy(po_hbm.at[pl.ds(r0, rows_per_tile)],
                                 po_sh.at[pl.ds(r0, rows_per_tile)], sem_g)
        for z in range(rows_per_tile // zrows):
            pltpu.sync_copy(zeros_v, cnt_sh.at[pl.ds(r0 + z * zrows, zrows)])
        tbl_s.wait()
        tbl_o.wait()
        plsc.subcore_barrier()

        def idx_load(ci, b):
            base = wid * per_w + ci * chunk
            pltpu.async_copy(sidx_hbm.at[pl.ds(base, chunk)], sidx_v.at[b],
                             sem_i[b])
            pltpu.async_copy(oidx_hbm.at[pl.ds(base, chunk)], oidx_v.at[b],
                             sem_i[b])

        def wait_i(b):
            pltpu.make_async_copy(sidx_hbm.at[pl.ds(0, chunk)], sidx_v.at[b],
                                  sem_i[b]).wait()
            pltpu.make_async_copy(oidx_hbm.at[pl.ds(0, chunk)], oidx_v.at[b],
                                  sem_i[b]).wait()

        def wait_w(b):
            pltpu.make_async_copy(rows_g.at[b], g_hbm.at[pl.ds(0, chunk)],
                                  sem_w[b]).wait()

        for b in range(nb):
            idx_load(b, b)

        def chunk_step(ci, b):
            base = wid * per_w + ci * chunk
            wait_i(b)

            @pl.when(ci >= nb)
            def _():
                wait_w(b)

            pltpu.sync_copy(ps_sh.at[sidx_v.at[b]], rows_g.at[b])
            pltpu.sync_copy(po_sh.at[oidx_v.at[b]], rows_g.at[b], add=True)
            pltpu.sync_copy(ones_v, cnt_sh.at[sidx_v.at[b]], add=True)
            pltpu.sync_copy(ones_v, cnt_sh.at[oidx_v.at[b]], add=True)
            pltpu.async_copy(rows_g.at[b], g_hbm.at[pl.ds(base, chunk)],
                             sem_w[b])
            ci2 = ci + nb
            ci2 = jnp.where(ci2 >= n_chunks, ci2 - n_chunks, ci2)
            idx_load(ci2, b)

        def chunk_body(ci, carry):
            for b in range(nb):
                @pl.when(lax.rem(ci, nb) == b)
                def _():
                    chunk_step(ci, b)
            return carry

        lax.fori_loop(0, n_chunks, chunk_body, 0)
        for b in range(nb):
            wait_i(b)
            wait_w(b)

        plsc.subcore_barrier()
        pltpu.sync_copy(cnt_sh.at[pl.ds(r0, rows_per_tile)],
                        cnt_hbm.at[cid, pl.ds(r0, rows_per_tile)])

    return gather_k(p_s, p_o, s_idx, o_idx)


def _sc_scatter(new_s, new_o, s_idx, o_idx, o, chunk):
    t, h = new_s.shape
    o_pad = _pad_rows(o)
    per_w = t // _NW
    n_chunks = per_w // chunk
    rows_per_tile = o_pad // _NS
    zrows = 128
    mesh = plsc.VectorSubcoreMesh(core_axis_name="c", subcore_axis_name="s")

    nb = 2

    @functools.partial(
        pl.kernel,
        out_type=jax.ShapeDtypeStruct((_NC, o_pad, h), jnp.float32),
        mesh=mesh,
        compiler_params=pltpu.CompilerParams(use_tc_tiling_on_sc=False),
        scratch_types=(
            pltpu.VMEM((nb, chunk), jnp.int32),
            pltpu.VMEM((nb, chunk), jnp.int32),
            pltpu.VMEM((nb, chunk, h), jnp.float32),
            pltpu.VMEM((nb, chunk, h), jnp.float32),
            pltpu.VMEM((zrows, h), jnp.float32),
            pltpu.VMEM_SHARED((o_pad, h), jnp.float32),
            [pltpu.SemaphoreType.DMA] * nb,
        ),
    )
    def scatter_k(ns_hbm, no_hbm, sidx_hbm, oidx_hbm, pooled_hbm,
                  sidx_v, oidx_v, rows_s, rows_o, zeros_v, pooled_sh,
                  sem_l):
        cid = lax.axis_index("c")
        sid = lax.axis_index("s")
        wid = sid * _NC + cid

        def fill_zeros(i, carry):
            for k in range(h // 16):
                zeros_v[i, pl.ds(k * 16, 16)] = jnp.zeros((16,), jnp.float32)
            return carry

        lax.fori_loop(0, zrows, fill_zeros, 0)

        r0 = sid * rows_per_tile
        for z in range(rows_per_tile // zrows):
            pltpu.sync_copy(zeros_v, pooled_sh.at[pl.ds(r0 + z * zrows, zrows)])
        plsc.subcore_barrier()

        def loads(ci, b):
            base = wid * per_w + ci * chunk
            pltpu.async_copy(sidx_hbm.at[pl.ds(base, chunk)], sidx_v.at[b],
                             sem_l[b])
            pltpu.async_copy(oidx_hbm.at[pl.ds(base, chunk)], oidx_v.at[b],
                             sem_l[b])
            pltpu.async_copy(ns_hbm.at[pl.ds(base, chunk)], rows_s.at[b],
                             sem_l[b])
            pltpu.async_copy(no_hbm.at[pl.ds(base, chunk)], rows_o.at[b],
                             sem_l[b])

        def wait_l(b):
            pltpu.make_async_copy(sidx_hbm.at[pl.ds(0, chunk)], sidx_v.at[b],
                                  sem_l[b]).wait()
            pltpu.make_async_copy(oidx_hbm.at[pl.ds(0, chunk)], oidx_v.at[b],
                                  sem_l[b]).wait()
            pltpu.make_async_copy(ns_hbm.at[pl.ds(0, chunk)], rows_s.at[b],
                                  sem_l[b]).wait()
            pltpu.make_async_copy(no_hbm.at[pl.ds(0, chunk)], rows_o.at[b],
                                  sem_l[b]).wait()

        for b in range(nb):
            loads(b, b)

        def chunk_step(ci, b):
            wait_l(b)
            pltpu.sync_copy(rows_s.at[b], pooled_sh.at[sidx_v.at[b]],
                            add=True)
            pltpu.sync_copy(rows_o.at[b], pooled_sh.at[oidx_v.at[b]],
                            add=True)
            ci2 = ci + nb
            ci2 = jnp.where(ci2 >= n_chunks, ci2 - n_chunks, ci2)
            loads(ci2, b)

        def chunk_body(ci, carry):
            for b in range(nb):
                @pl.when(lax.rem(ci, nb) == b)
                def _():
                    chunk_step(ci, b)
            return carry

        lax.fori_loop(0, n_chunks, chunk_body, 0)
        for b in range(nb):
            wait_l(b)

        plsc.subcore_barrier()
        pltpu.sync_copy(pooled_sh.at[pl.ds(r0, rows_per_tile)],
                        pooled_hbm.at[cid, pl.ds(r0, rows_per_tile)])

    return scatter_k(new_s, new_o, s_idx, o_idx)


def kernel(obj_vecs, pred_vecs, edges, W1, b1, W2, b2, W3, b3, W4, b4):
    o, din = obj_vecs.shape
    h = W1.shape[0]
    dout = W4.shape[0]

    s_idx = edges[:, 0]
    o_idx = edges[:, 1]
    w1s_t = W1[:, :din].T
    w1p_t = W1[:, din:2 * din].T
    w1o_t = W1[:, 2 * din:].T
    b1r = b1.reshape(1, h)
    w2_t = W2.T
    b2r = b2.reshape(1, -1)
    w3_t = W3.T
    b3r = b3.reshape(1, h)
    w4_t = W4.T
    b4r = b4.reshape(1, dout)

    o_pad = _pad_rows(o)
    obj_pad = jnp.pad(obj_vecs, ((0, o_pad - o), (0, 0)))
    p_s, p_o = _tc_proj(obj_pad, w1s_t, w1o_t)
    g, cnt = _sc_gather(p_s, p_o, s_idx, o_idx, 200)
    new_s, new_p, new_o = _tc_edge_mlp(g, pred_vecs, w1p_t, b1r, w2_t, b2r)
    pooled = _sc_scatter(new_s, new_o, s_idx, o_idx, o, 200)
    new_obj = _tc_final([pooled], [cnt], w3_t, b3r, w4_t, b4r, o)
    return (new_obj, new_p)


# speedup vs baseline: 1.4384x; 1.0036x over previous
import functools

import jax
import jax.numpy as jnp
from jax import lax
from jax.experimental import pallas as pl
from jax.experimental.pallas import tpu as pltpu
from jax.experimental.pallas import tpu_sc as plsc

_NC = 2
_NS = 16
_NW = _NC * _NS
_CNT_W = 16


def _proj_body(obj_ref, w1s_ref, w1o_ref, ps_ref, po_ref):
    x = obj_ref[...]
    ps_ref[...] = jnp.dot(x, w1s_ref[...], preferred_element_type=jnp.float32)
    po_ref[...] = jnp.dot(x, w1o_ref[...], preferred_element_type=jnp.float32)


def _tc_proj(obj, w1s_t, w1o_t):
    o, din = obj.shape
    h = w1s_t.shape[1]
    bo = 2048
    return pl.pallas_call(
        _proj_body,
        grid=(o // bo,),
        in_specs=[
            pl.BlockSpec((bo, din), lambda i: (i, 0)),
            pl.BlockSpec((din, h), lambda i: (0, 0)),
            pl.BlockSpec((din, h), lambda i: (0, 0)),
        ],
        out_specs=[pl.BlockSpec((bo, h), lambda i: (i, 0))] * 2,
        out_shape=[jax.ShapeDtypeStruct((o, h), jnp.float32)] * 2,
    )(obj, w1s_t, w1o_t)


def _edge_body(g_ref, pred_ref, w1p_ref, b1_ref, w2_ref, b2_ref,
               ns_ref, np_ref, no_ref):
    h = jnp.dot(pred_ref[...], w1p_ref[...], preferred_element_type=jnp.float32)
    h = jnp.maximum(h + g_ref[...] + b1_ref[...], 0.0)
    t = jnp.dot(h, w2_ref[...], preferred_element_type=jnp.float32)
    t = jnp.maximum(t + b2_ref[...], 0.0)
    hh = ns_ref.shape[1]
    dout = np_ref.shape[1]
    ns_ref[...] = t[:, :hh]
    np_ref[...] = t[:, hh:hh + dout]
    no_ref[...] = t[:, hh + dout:]


def _tc_edge_mlp(g, pred, w1p_t, b1r, w2_t, b2r):
    t, din = pred.shape
    h = g.shape[1]
    dout2 = w2_t.shape[1]
    dout = dout2 - 2 * h
    be = 8000
    return pl.pallas_call(
        _edge_body,
        grid=(t // be,),
        in_specs=[
            pl.BlockSpec((be, h), lambda i: (i, 0)),
            pl.BlockSpec((be, din), lambda i: (i, 0)),
            pl.BlockSpec((din, h), lambda i: (0, 0)),
            pl.BlockSpec((1, h), lambda i: (0, 0)),
            pl.BlockSpec((h, dout2), lambda i: (0, 0)),
            pl.BlockSpec((1, dout2), lambda i: (0, 0)),
        ],
        out_specs=[
            pl.BlockSpec((be, h), lambda i: (i, 0)),
            pl.BlockSpec((be, dout), lambda i: (i, 0)),
            pl.BlockSpec((be, h), lambda i: (i, 0)),
        ],
        out_shape=[
            jax.ShapeDtypeStruct((t, h), jnp.float32),
            jax.ShapeDtypeStruct((t, dout), jnp.float32),
            jax.ShapeDtypeStruct((t, h), jnp.float32),
        ],
    )(g, pred, w1p_t, b1r, w2_t, b2r)


def _final_body(*refs):
    n_parts = (len(refs) - 5) // 2
    pp_refs = refs[:n_parts]
    cc_refs = refs[n_parts:2 * n_parts]
    w3_ref, b3_ref, w4_ref, b4_ref, out_ref = refs[2 * n_parts:]
    p = pp_refs[0][0] + pp_refs[0][1]
    c = cc_refs[0][0, :, 0:1] + cc_refs[0][1, :, 0:1]
    for k in range(1, n_parts):
        p = p + pp_refs[k][0] + pp_refs[k][1]
        c = c + cc_refs[k][0, :, 0:1] + cc_refs[k][1, :, 0:1]
    p = p / jnp.maximum(c, 1.0)
    h = jnp.dot(p, w3_ref[...], preferred_element_type=jnp.float32)
    h = jnp.maximum(h + b3_ref[...], 0.0)
    y = jnp.dot(h, w4_ref[...], preferred_element_type=jnp.float32)
    out_ref[...] = jnp.maximum(y + b4_ref[...], 0.0)


def _tc_final(pooled_list, cnt_list, w3_t, b3r, w4_t, b4r, o):
    h = pooled_list[0].shape[2]
    dout = w4_t.shape[1]
    bo = 2000
    n_parts = len(pooled_list)
    return pl.pallas_call(
        _final_body,
        grid=(o // bo,),
        in_specs=(
            [pl.BlockSpec((2, bo, h), lambda i: (0, i, 0))] * n_parts
            + [pl.BlockSpec((2, bo, _CNT_W), lambda i: (0, i, 0))] * n_parts
            + [
                pl.BlockSpec((h, h), lambda i: (0, 0)),
                pl.BlockSpec((1, h), lambda i: (0, 0)),
                pl.BlockSpec((h, dout), lambda i: (0, 0)),
                pl.BlockSpec((1, dout), lambda i: (0, 0)),
            ]
        ),
        out_specs=pl.BlockSpec((bo, dout), lambda i: (i, 0)),
        out_shape=jax.ShapeDtypeStruct((o, dout), jnp.float32),
    )(*pooled_list, *cnt_list, w3_t, b3r, w4_t, b4r)


def _pad_rows(o):
    return -(-o // (_NS * 128)) * (_NS * 128)


def _sc_gather(p_s, p_o, s_idx, o_idx, chunk):
    o, h = p_s.shape
    t = s_idx.shape[0]
    o_pad = _pad_rows(o)
    per_w = t // _NW
    n_chunks = per_w // chunk
    rows_per_tile = o_pad // _NS
    zrows = 128
    mesh = plsc.VectorSubcoreMesh(core_axis_name="c", subcore_axis_name="s")

    nb = 2

    @functools.partial(
        pl.kernel,
        out_type=(
            jax.ShapeDtypeStruct((t, h), jnp.float32),
            jax.ShapeDtypeStruct((_NC, o_pad, _CNT_W), jnp.float32),
        ),
        mesh=mesh,
        compiler_params=pltpu.CompilerParams(use_tc_tiling_on_sc=False),
        scratch_types=(
            pltpu.VMEM((nb, chunk), jnp.int32),
            pltpu.VMEM((nb, chunk), jnp.int32),
            pltpu.VMEM((nb, chunk, h), jnp.float32),
            pltpu.VMEM((chunk, _CNT_W), jnp.float32),
            pltpu.VMEM((zrows, _CNT_W), jnp.float32),
            pltpu.VMEM_SHARED((o_pad, _CNT_W), jnp.float32),
            pltpu.VMEM_SHARED((o_pad, 64), jnp.float32),
            pltpu.VMEM_SHARED((o_pad, 64), jnp.float32),
            [pltpu.SemaphoreType.DMA] * nb,
            pltpu.SemaphoreType.DMA,
            [pltpu.SemaphoreType.DMA] * nb,
        ),
    )
    def gather_k(ps_hbm, po_hbm, sidx_hbm, oidx_hbm,
                 g_hbm, cnt_hbm,
                 sidx_v, oidx_v, rows_g,
                 ones_v, zeros_v, cnt_sh, ps_sh, po_sh, sem_i, sem_g, sem_w):
        cid = lax.axis_index("c")
        sid = lax.axis_index("s")
        wid = sid * _NC + cid

        def fill_ones(i, carry):
            ones_v[i, :] = jnp.full((16,), 1.0, jnp.float32)
            return carry

        lax.fori_loop(0, chunk, fill_ones, 0)

        def fill_zeros(i, carry):
            zeros_v[i, :] = jnp.zeros((16,), jnp.float32)
            return carry

        lax.fori_loop(0, zrows, fill_zeros, 0)

        r0 = sid * rows_per_tile
        tbl_s = pltpu.async_copy(ps_hbm.at[pl.ds(r0, rows_per_tile)],
                                 ps_sh.at[pl.ds(r0, rows_per_tile)], sem_g)
        tbl_o = pltpu.async_copy(po_hbm.at[pl.ds(r0, rows_per_tile)],
                                 po_sh.at[pl.ds(r0, rows_per_tile)], sem_g)
        for z in range(rows_per_tile // zrows):
            pltpu.sync_copy(zeros_v, cnt_sh.at[pl.ds(r0 + z * zrows, zrows)])
        tbl_s.wait()
        tbl_o.wait()
        plsc.subcore_barrier()

        def idx_load(ci, b):
            base = wid * per_w + ci * chunk
            pltpu.async_copy(sidx_hbm.at[pl.ds(base, chunk)], sidx_v.at[b],
                             sem_i[b])
            pltpu.async_copy(oidx_hbm.at[pl.ds(base, chunk)], oidx_v.at[b],
                             sem_i[b])

        def wait_i(b):
            pltpu.make_async_copy(sidx_hbm.at[pl.ds(0, chunk)], sidx_v.at[b],
                                  sem_i[b]).wait()
            pltpu.make_async_copy(oidx_hbm.at[pl.ds(0, chunk)], oidx_v.at[b],
                                  sem_i[b]).wait()

        def wait_w(b):
            pltpu.make_async_copy(rows_g.at[b], g_hbm.at[pl.ds(0, chunk)],
                                  sem_w[b]).wait()

        for b in range(nb):
            idx_load(b, b)

        def chunk_step(ci, b):
            base = wid * per_w + ci * chunk
            wait_i(b)

            @pl.when(ci >= nb)
            def _():
                wait_w(b)

            pltpu.sync_copy(ps_sh.at[sidx_v.at[b]], rows_g.at[b])
            pltpu.sync_copy(po_sh.at[oidx_v.at[b]], rows_g.at[b], add=True)
            pltpu.sync_copy(ones_v, cnt_sh.at[sidx_v.at[b]], add=True)
            pltpu.sync_copy(ones_v, cnt_sh.at[oidx_v.at[b]], add=True)
            pltpu.async_copy(rows_g.at[b], g_hbm.at[pl.ds(base, chunk)],
                             sem_w[b])
            ci2 = ci + nb
            ci2 = jnp.where(ci2 >= n_chunks, ci2 - n_chunks, ci2)
            idx_load(ci2, b)

        def chunk_body(ci, carry):
            for b in range(nb):
                @pl.when(lax.rem(ci, nb) == b)
                def _():
                    chunk_step(ci, b)
            return carry

        lax.fori_loop(0, n_chunks, chunk_body, 0)
        for b in range(nb):
            wait_i(b)
            wait_w(b)

        plsc.subcore_barrier()
        pltpu.sync_copy(cnt_sh.at[pl.ds(r0, rows_per_tile)],
                        cnt_hbm.at[cid, pl.ds(r0, rows_per_tile)])

    return gather_k(p_s, p_o, s_idx, o_idx)


def _sc_scatter(new_s, new_o, s_idx, o_idx, o, chunk):
    t, h = new_s.shape
    o_pad = _pad_rows(o)
    per_w = t // _NW
    n_chunks = per_w // chunk
    rows_per_tile = o_pad // _NS
    zrows = 128
    mesh = plsc.VectorSubcoreMesh(core_axis_name="c", subcore_axis_name="s")

    nb = 2

    @functools.partial(
        pl.kernel,
        out_type=jax.ShapeDtypeStruct((_NC, o_pad, h), jnp.float32),
        mesh=mesh,
        compiler_params=pltpu.CompilerParams(use_tc_tiling_on_sc=False),
        scratch_types=(
            pltpu.VMEM((nb, chunk), jnp.int32),
            pltpu.VMEM((nb, chunk), jnp.int32),
            pltpu.VMEM((nb, chunk, h), jnp.float32),
            pltpu.VMEM((nb, chunk, h), jnp.float32),
            pltpu.VMEM((zrows, h), jnp.float32),
            pltpu.VMEM_SHARED((o_pad, h), jnp.float32),
            [pltpu.SemaphoreType.DMA] * nb,
        ),
    )
    def scatter_k(ns_hbm, no_hbm, sidx_hbm, oidx_hbm, pooled_hbm,
                  sidx_v, oidx_v, rows_s, rows_o, zeros_v, pooled_sh,
                  sem_l):
        cid = lax.axis_index("c")
        sid = lax.axis_index("s")
        wid = sid * _NC + cid

        def fill_zeros(i, carry):
            for k in range(h // 16):
                zeros_v[i, pl.ds(k * 16, 16)] = jnp.zeros((16,), jnp.float32)
            return carry

        lax.fori_loop(0, zrows, fill_zeros, 0)

        r0 = sid * rows_per_tile
        for z in range(rows_per_tile // zrows):
            pltpu.sync_copy(zeros_v, pooled_sh.at[pl.ds(r0 + z * zrows, zrows)])
        plsc.subcore_barrier()

        def loads(ci, b):
            base = wid * per_w + ci * chunk
            pltpu.async_copy(sidx_hbm.at[pl.ds(base, chunk)], sidx_v.at[b],
                             sem_l[b])
            pltpu.async_copy(oidx_hbm.at[pl.ds(base, chunk)], oidx_v.at[b],
                             sem_l[b])
            pltpu.async_copy(ns_hbm.at[pl.ds(base, chunk)], rows_s.at[b],
                             sem_l[b])
            pltpu.async_copy(no_hbm.at[pl.ds(base, chunk)], rows_o.at[b],
                             sem_l[b])

        def wait_l(b):
            pltpu.make_async_copy(sidx_hbm.at[pl.ds(0, chunk)], sidx_v.at[b],
                                  sem_l[b]).wait()
            pltpu.make_async_copy(oidx_hbm.at[pl.ds(0, chunk)], oidx_v.at[b],
                                  sem_l[b]).wait()
            pltpu.make_async_copy(ns_hbm.at[pl.ds(0, chunk)], rows_s.at[b],
                                  sem_l[b]).wait()
            pltpu.make_async_copy(no_hbm.at[pl.ds(0, chunk)], rows_o.at[b],
                                  sem_l[b]).wait()

        for b in range(nb):
            loads(b, b)

        def chunk_step(ci, b):
            wait_l(b)
            pltpu.sync_copy(rows_s.at[b], pooled_sh.at[sidx_v.at[b]],
                            add=True)
            pltpu.sync_copy(rows_o.at[b], pooled_sh.at[oidx_v.at[b]],
                            add=True)
            ci2 = ci + nb
            ci2 = jnp.where(ci2 >= n_chunks, ci2 - n_chunks, ci2)
            loads(ci2, b)

        def chunk_body(ci, carry):
            for b in range(nb):
                @pl.when(lax.rem(ci, nb) == b)
                def _():
                    chunk_step(ci, b)
            return carry

        lax.fori_loop(0, n_chunks, chunk_body, 0)
        for b in range(nb):
            wait_l(b)

        plsc.subcore_barrier()
        pltpu.sync_copy(pooled_sh.at[pl.ds(r0, rows_per_tile)],
                        pooled_hbm.at[cid, pl.ds(r0, rows_per_tile)])

    return scatter_k(new_s, new_o, s_idx, o_idx)


def kernel(obj_vecs, pred_vecs, edges, W1, b1, W2, b2, W3, b3, W4, b4):
    o, din = obj_vecs.shape
    h = W1.shape[0]
    dout = W4.shape[0]

    s_idx = edges[:, 0]
    o_idx = edges[:, 1]
    w1s_t = W1[:, :din].T
    w1p_t = W1[:, din:2 * din].T
    w1o_t = W1[:, 2 * din:].T
    b1r = b1.reshape(1, h)
    w2_t = W2.T
    b2r = b2.reshape(1, -1)
    w3_t = W3.T
    b3r = b3.reshape(1, h)
    w4_t = W4.T
    b4r = b4.reshape(1, dout)

    o_pad = _pad_rows(o)
    obj_pad = jnp.pad(obj_vecs, ((0, o_pad - o), (0, 0)))
    p_s, p_o = _tc_proj(obj_pad, w1s_t, w1o_t)
    g, cnt = _sc_gather(p_s, p_o, s_idx, o_idx, 200)
    new_s, new_p, new_o = _tc_edge_mlp(g, pred_vecs, w1p_t, b1r, w2_t, b2r)
    pooled = _sc_scatter(new_s, new_o, s_idx, o_idx, o, 200)
    new_obj = _tc_final([pooled], [cnt], w3_t, b3r, w4_t, b4r, o)
    return (new_obj, new_p)
